# initial kernel scaffold (unmeasured)
import jax
import jax.numpy as jnp
from jax import lax
from jax.experimental import pallas as pl
from jax.experimental.pallas import tpu as pltpu

N_DEV = 16
SQ = 256
SKV = 4096
H_LOC = 8
DH = 128
D_MODEL = 1024
D_LOC = H_LOC * DH
CHUNK = SQ // N_DEV
SCALE = 0.08838834764831843


def kernel(x, Wq, K_ext, V_ext, Wo):
    i = lax.axis_index("i")
    x2 = x[0].astype(jnp.bfloat16)
    Wq_s = lax.dynamic_slice(Wq, (0, i * D_LOC), (D_MODEL, D_LOC))
    Wo_s = lax.dynamic_slice(Wo, (i * D_LOC, 0), (D_LOC, D_MODEL))
    Wq_s = Wq_s.astype(jnp.bfloat16)
    Wo_s = Wo_s.astype(jnp.bfloat16)
    K = K_ext[0]
    V = V_ext[0]

    def body(x_ref, wq_ref, k_ref, v_ref, wo_ref, out_ref,
             acc_ref, rs_buf, rs_send, rs_recv, ag_send, ag_recv):
        my = lax.axis_index("i")
        right = lax.rem(my + 1, N_DEV)

        q = jnp.dot(x_ref[...], wq_ref[...],
                    preferred_element_type=jnp.float32)
        qb = lax.broadcasted_iota(jnp.int32, (SQ, SKV), 0) // 64
        kb = lax.broadcasted_iota(jnp.int32, (SQ, SKV), 1) // 64
        mask = (qb == kb) | (kb == 0) | ((qb + kb) % 3 == 0)
        ctx_parts = []
        for h in range(H_LOC):
            q_h = q[:, h * DH:(h + 1) * DH].astype(jnp.bfloat16)
            k_h = k_ref[:, h, :].astype(jnp.bfloat16)
            s = lax.dot_general(q_h, k_h, (((1,), (1,)), ((), ())),
                                preferred_element_type=jnp.float32)
            s = jnp.where(mask, s * SCALE, -1e9)
            m = jnp.max(s, axis=-1, keepdims=True)
            w = jnp.exp(s - m)
            w = w / jnp.sum(w, axis=-1, keepdims=True)
            v_h = v_ref[:, h, :].astype(jnp.bfloat16)
            ctx_parts.append(jnp.dot(w.astype(jnp.bfloat16), v_h,
                                     preferred_element_type=jnp.float32))
        ctx = jnp.concatenate(ctx_parts, axis=-1).astype(jnp.bfloat16)
        partial = jnp.dot(ctx, wo_ref[...],
                          preferred_element_type=jnp.float32)
        acc_ref[...] = partial.reshape(N_DEV, CHUNK, D_MODEL)

        for s_i in range(N_DEV - 1):
            sc = lax.rem(my - s_i + N_DEV, N_DEV)
            rdma = pltpu.make_async_remote_copy(
                src_ref=acc_ref.at[sc],
                dst_ref=rs_buf.at[s_i],
                send_sem=rs_send.at[s_i],
                recv_sem=rs_recv.at[s_i],
                device_id=(right,),
                device_id_type=pl.DeviceIdType.MESH,
            )
            rdma.start()
            rdma.wait()
            rc = lax.rem(my - s_i - 1 + N_DEV, N_DEV)
            acc_ref[rc] = acc_ref[rc] + rs_buf[s_i]

        for s_i in range(N_DEV - 1):
            gc = lax.rem(my + 1 - s_i + N_DEV, N_DEV)
            rdma = pltpu.make_async_remote_copy(
                src_ref=acc_ref.at[gc],
                dst_ref=acc_ref.at[gc],
                send_sem=ag_send.at[s_i],
                recv_sem=ag_recv.at[s_i],
                device_id=(right,),
                device_id_type=pl.DeviceIdType.MESH,
            )
            rdma.start()
            rdma.wait()

        out_ref[...] = acc_ref[...].reshape(1, SQ, D_MODEL)

    return pl.pallas_call(
        body,
        out_shape=jax.ShapeDtypeStruct((1, SQ, D_MODEL), jnp.float32),
        in_specs=[pl.BlockSpec(memory_space=pltpu.VMEM)] * 5,
        out_specs=pl.BlockSpec(memory_space=pltpu.VMEM),
        scratch_shapes=[
            pltpu.VMEM((N_DEV, CHUNK, D_MODEL), jnp.float32),
            pltpu.VMEM((N_DEV - 1, CHUNK, D_MODEL), jnp.float32),
            pltpu.SemaphoreType.DMA((N_DEV - 1,)),
            pltpu.SemaphoreType.DMA((N_DEV - 1,)),
            pltpu.SemaphoreType.DMA((N_DEV - 1,)),
            pltpu.SemaphoreType.DMA((N_DEV - 1,)),
        ],
        compiler_params=pltpu.CompilerParams(collective_id=0),
    )(x2, Wq_s, K, V, Wo_s)


# baseline (device time: 171299 ns/iter reference)
import jax
import jax.numpy as jnp
from jax import lax
from jax.experimental import pallas as pl
from jax.experimental.pallas import tpu as pltpu

N_DEV = 16
SQ = 256
SKV = 4096
H_LOC = 8
DH = 128
D_MODEL = 1024
D_LOC = H_LOC * DH
CHUNK = SQ // N_DEV
SCALE = 0.08838834764831843


def kernel(x, Wq, K_ext, V_ext, Wo):
    i = lax.axis_index("i")
    x2 = x[0].astype(jnp.bfloat16)
    Wq_s = lax.dynamic_slice(Wq, (0, i * D_LOC), (D_MODEL, D_LOC))
    Wo_s = lax.dynamic_slice(Wo, (i * D_LOC, 0), (D_LOC, D_MODEL))
    Wq_s = Wq_s.astype(jnp.bfloat16)
    Wo_s = Wo_s.astype(jnp.bfloat16)
    K = K_ext[0]
    V = V_ext[0]

    def body(x_ref, wq_ref, k_ref, v_ref, wo_ref, out_ref,
             acc_ref, rs_buf, rs_send, rs_recv, ag_send, ag_recv):
        my = lax.axis_index("i")
        right = lax.rem(my + 1, N_DEV)

        q = jnp.dot(x_ref[...], wq_ref[...],
                    preferred_element_type=jnp.float32)
        qb = lax.broadcasted_iota(jnp.int32, (SQ, SKV), 0) // 64
        kb = lax.broadcasted_iota(jnp.int32, (SQ, SKV), 1) // 64
        mask = (qb == kb) | (kb == 0) | ((qb + kb) % 3 == 0)
        ctx_parts = []
        for h in range(H_LOC):
            q_h = q[:, h * DH:(h + 1) * DH].astype(jnp.bfloat16)
            k_h = k_ref[:, h, :].astype(jnp.bfloat16)
            s = lax.dot_general(q_h, k_h, (((1,), (1,)), ((), ())),
                                preferred_element_type=jnp.float32)
            s = jnp.where(mask, s * SCALE, -1e9)
            m = jnp.max(s, axis=-1, keepdims=True)
            w = jnp.exp(s - m)
            w = w / jnp.sum(w, axis=-1, keepdims=True)
            v_h = v_ref[:, h, :].astype(jnp.bfloat16)
            ctx_parts.append(jnp.dot(w.astype(jnp.bfloat16), v_h,
                                     preferred_element_type=jnp.float32))
        ctx = jnp.concatenate(ctx_parts, axis=-1).astype(jnp.bfloat16)
        partial = jnp.dot(ctx, wo_ref[...],
                          preferred_element_type=jnp.float32)
        acc_ref[...] = partial.reshape(N_DEV, CHUNK, D_MODEL)

        for s_i in range(N_DEV - 1):
            sc = lax.rem(my - s_i + N_DEV, N_DEV)
            rdma = pltpu.make_async_remote_copy(
                src_ref=acc_ref.at[sc],
                dst_ref=rs_buf.at[s_i],
                send_sem=rs_send.at[s_i],
                recv_sem=rs_recv.at[s_i],
                device_id=(right,),
                device_id_type=pl.DeviceIdType.MESH,
            )
            rdma.start()
            rdma.wait()
            rc = lax.rem(my - s_i - 1 + N_DEV, N_DEV)
            acc_ref[rc] = acc_ref[rc] + rs_buf[s_i]

        for s_i in range(N_DEV - 1):
            gc = lax.rem(my + 1 - s_i + N_DEV, N_DEV)
            rdma = pltpu.make_async_remote_copy(
                src_ref=acc_ref.at[gc],
                dst_ref=acc_ref.at[gc],
                send_sem=ag_send.at[s_i],
                recv_sem=ag_recv.at[s_i],
                device_id=(right,),
                device_id_type=pl.DeviceIdType.MESH,
            )
            rdma.start()
            rdma.wait()

        out_ref[...] = acc_ref[...].reshape(1, SQ, D_MODEL)

    return pl.pallas_call(
        body,
        out_shape=jax.ShapeDtypeStruct((1, SQ, D_MODEL), jnp.float32),
        in_specs=[pl.BlockSpec(memory_space=pltpu.VMEM)] * 5,
        out_specs=pl.BlockSpec(memory_space=pltpu.VMEM),
        scratch_shapes=[
            pltpu.VMEM((N_DEV, CHUNK, D_MODEL), jnp.float32),
            pltpu.VMEM((N_DEV - 1, CHUNK, D_MODEL), jnp.float32),
            pltpu.SemaphoreType.DMA((N_DEV - 1,)),
            pltpu.SemaphoreType.DMA((N_DEV - 1,)),
            pltpu.SemaphoreType.DMA((N_DEV - 1,)),
            pltpu.SemaphoreType.DMA((N_DEV - 1,)),
        ],
    )(x2, Wq_s, K, V, Wo_s)


# device time: 140877 ns/iter; 1.2159x vs baseline; 1.2159x over previous
import jax
import jax.numpy as jnp
from jax import lax
from jax.experimental import pallas as pl
from jax.experimental.pallas import tpu as pltpu

N_DEV = 16
SQ = 256
SKV = 4096
H_LOC = 8
DH = 128
D_MODEL = 1024
D_LOC = H_LOC * DH
CHUNK = SQ // N_DEV
SCALE = 0.08838834764831843


def kernel(x, Wq, K_ext, V_ext, Wo):
    i = lax.axis_index("i")
    x2 = x[0].astype(jnp.bfloat16)
    Wq_s = lax.dynamic_slice(Wq, (0, i * D_LOC), (D_MODEL, D_LOC))
    Wo_s = lax.dynamic_slice(Wo, (i * D_LOC, 0), (D_LOC, D_MODEL))
    Wq_s = Wq_s.astype(jnp.bfloat16)
    Wo_s = Wo_s.astype(jnp.bfloat16)
    K = K_ext[0]
    V = V_ext[0]

    def body(x_ref, wq_ref, k_ref, v_ref, wo_ref, out_ref,
             acc_ref, rs_buf, rs_send, rs_recv, ag_send, ag_recv):
        my = lax.axis_index("i")
        right = lax.rem(my + 1, N_DEV)

        q = jnp.dot(x_ref[...], wq_ref[...],
                    preferred_element_type=jnp.float32)
        qb = lax.broadcasted_iota(jnp.int32, (SQ, SKV), 0) // 64
        kb = lax.broadcasted_iota(jnp.int32, (SQ, SKV), 1) // 64
        mask = (qb == kb) | (kb == 0) | ((qb + kb) % 3 == 0)
        ctx_parts = []
        for h in range(H_LOC):
            q_h = q[:, h * DH:(h + 1) * DH].astype(jnp.bfloat16)
            k_h = k_ref[:, h, :].astype(jnp.bfloat16)
            s = lax.dot_general(q_h, k_h, (((1,), (1,)), ((), ())),
                                preferred_element_type=jnp.float32)
            s = jnp.where(mask, s * SCALE, -1e9)
            m = jnp.max(s, axis=-1, keepdims=True)
            w = jnp.exp(s - m)
            w = w / jnp.sum(w, axis=-1, keepdims=True)
            v_h = v_ref[:, h, :].astype(jnp.bfloat16)
            ctx_parts.append(jnp.dot(w.astype(jnp.bfloat16), v_h,
                                     preferred_element_type=jnp.float32))
        ctx = jnp.concatenate(ctx_parts, axis=-1).astype(jnp.bfloat16)
        partial = jnp.dot(ctx, wo_ref[...],
                          preferred_element_type=jnp.float32)
        acc_ref[...] = partial.reshape(N_DEV, CHUNK, D_MODEL)

        stage_off = 0
        for k, b in enumerate((8, 4, 2, 1)):
            partner = my ^ b
            keep_start = my & (N_DEV - b)
            send_start = keep_start ^ b
            rdma = pltpu.make_async_remote_copy(
                src_ref=acc_ref.at[pl.ds(send_start, b)],
                dst_ref=rs_buf.at[pl.ds(stage_off, b)],
                send_sem=rs_send.at[k],
                recv_sem=rs_recv.at[k],
                device_id=(partner,),
                device_id_type=pl.DeviceIdType.MESH,
            )
            rdma.start()
            rdma.wait()
            acc_ref[pl.ds(keep_start, b)] = (
                acc_ref[pl.ds(keep_start, b)] + rs_buf[pl.ds(stage_off, b)]
            )
            stage_off += b

        for k, b in enumerate((1, 2, 4, 8)):
            partner = my ^ b
            seg_start = my & (N_DEV - b)
            rdma = pltpu.make_async_remote_copy(
                src_ref=acc_ref.at[pl.ds(seg_start, b)],
                dst_ref=acc_ref.at[pl.ds(seg_start, b)],
                send_sem=ag_send.at[k],
                recv_sem=ag_recv.at[k],
                device_id=(partner,),
                device_id_type=pl.DeviceIdType.MESH,
            )
            rdma.start()
            rdma.wait()

        out_ref[...] = acc_ref[...].reshape(1, SQ, D_MODEL)

    return pl.pallas_call(
        body,
        out_shape=jax.ShapeDtypeStruct((1, SQ, D_MODEL), jnp.float32),
        in_specs=[pl.BlockSpec(memory_space=pltpu.VMEM)] * 5,
        out_specs=pl.BlockSpec(memory_space=pltpu.VMEM),
        scratch_shapes=[
            pltpu.VMEM((N_DEV, CHUNK, D_MODEL), jnp.float32),
            pltpu.VMEM((N_DEV - 1, CHUNK, D_MODEL), jnp.float32),
            pltpu.SemaphoreType.DMA((4,)),
            pltpu.SemaphoreType.DMA((4,)),
            pltpu.SemaphoreType.DMA((4,)),
            pltpu.SemaphoreType.DMA((4,)),
        ],
    )(x2, Wq_s, K, V, Wo_s)


# device time: 106749 ns/iter; 1.6047x vs baseline; 1.3197x over previous
import jax
import jax.numpy as jnp
from jax import lax
from jax.experimental import pallas as pl
from jax.experimental.pallas import tpu as pltpu

N_DEV = 16
SQ = 256
SKV = 4096
H_LOC = 8
DH = 128
D_MODEL = 1024
D_LOC = H_LOC * DH
CHUNK = SQ // N_DEV
SCALE = 0.08838834764831843


def kernel(x, Wq, K_ext, V_ext, Wo):
    i = lax.axis_index("i")
    x2 = x[0].astype(jnp.bfloat16)
    Wq_s = lax.dynamic_slice(Wq, (0, i * D_LOC), (D_MODEL, D_LOC))
    Wo_s = lax.dynamic_slice(Wo, (i * D_LOC, 0), (D_LOC, D_MODEL))
    Wq_s = (Wq_s * SCALE).astype(jnp.bfloat16)
    Wo_s = Wo_s.astype(jnp.bfloat16)
    K = K_ext[0].reshape(SKV, H_LOC * DH)
    V = V_ext[0].reshape(SKV, H_LOC * DH)

    def body(x_ref, wq_ref, k_ref, v_ref, wo_ref, out_ref,
             acc_ref, rs_buf, rs_send, rs_recv, ag_send, ag_recv):
        my = lax.axis_index("i")
        right = lax.rem(my + 1, N_DEV)

        q = jnp.dot(x_ref[...], wq_ref[...],
                    preferred_element_type=jnp.float32)
        qb = lax.broadcasted_iota(jnp.int32, (SQ, SKV), 0) // 64
        kb = lax.broadcasted_iota(jnp.int32, (SQ, SKV), 1) // 64
        mask = (qb == kb) | (kb == 0) | ((qb + kb) % 3 == 0)
        bias = jnp.where(mask, 0.0, -1e9).astype(jnp.float32)
        ctx_parts = []
        for h in range(H_LOC):
            q_h = q[:, h * DH:(h + 1) * DH].astype(jnp.bfloat16)
            k_h = k_ref[:, h * DH:(h + 1) * DH].astype(jnp.bfloat16)
            s = lax.dot_general(q_h, k_h, (((1,), (1,)), ((), ())),
                                preferred_element_type=jnp.float32)
            w = jnp.exp(s + bias)
            denom = jnp.sum(w, axis=-1, keepdims=True)
            v_h = v_ref[:, h * DH:(h + 1) * DH].astype(jnp.bfloat16)
            ctx_h = jnp.dot(w.astype(jnp.bfloat16), v_h,
                            preferred_element_type=jnp.float32)
            ctx_parts.append(ctx_h / denom)
        ctx = jnp.concatenate(ctx_parts, axis=-1).astype(jnp.bfloat16)
        partial = jnp.dot(ctx, wo_ref[...],
                          preferred_element_type=jnp.float32)
        acc_ref[...] = partial.reshape(N_DEV, CHUNK, D_MODEL)

        stage_off = 0
        for k, b in enumerate((8, 4, 2, 1)):
            partner = my ^ b
            keep_start = my & (N_DEV - b)
            send_start = keep_start ^ b
            rdma = pltpu.make_async_remote_copy(
                src_ref=acc_ref.at[pl.ds(send_start, b)],
                dst_ref=rs_buf.at[pl.ds(stage_off, b)],
                send_sem=rs_send.at[k],
                recv_sem=rs_recv.at[k],
                device_id=(partner,),
                device_id_type=pl.DeviceIdType.MESH,
            )
            rdma.start()
            rdma.wait()
            acc_ref[pl.ds(keep_start, b)] = (
                acc_ref[pl.ds(keep_start, b)] + rs_buf[pl.ds(stage_off, b)]
            )
            stage_off += b

        for k, b in enumerate((1, 2, 4, 8)):
            partner = my ^ b
            seg_start = my & (N_DEV - b)
            rdma = pltpu.make_async_remote_copy(
                src_ref=acc_ref.at[pl.ds(seg_start, b)],
                dst_ref=acc_ref.at[pl.ds(seg_start, b)],
                send_sem=ag_send.at[k],
                recv_sem=ag_recv.at[k],
                device_id=(partner,),
                device_id_type=pl.DeviceIdType.MESH,
            )
            rdma.start()
            rdma.wait()

        out_ref[...] = acc_ref[...].reshape(1, SQ, D_MODEL)

    return pl.pallas_call(
        body,
        out_shape=jax.ShapeDtypeStruct((1, SQ, D_MODEL), jnp.float32),
        in_specs=[pl.BlockSpec(memory_space=pltpu.VMEM)] * 5,
        out_specs=pl.BlockSpec(memory_space=pltpu.VMEM),
        scratch_shapes=[
            pltpu.VMEM((N_DEV, CHUNK, D_MODEL), jnp.float32),
            pltpu.VMEM((N_DEV - 1, CHUNK, D_MODEL), jnp.float32),
            pltpu.SemaphoreType.DMA((4,)),
            pltpu.SemaphoreType.DMA((4,)),
            pltpu.SemaphoreType.DMA((4,)),
            pltpu.SemaphoreType.DMA((4,)),
        ],
    )(x2, Wq_s, K, V, Wo_s)


# device time: 97061 ns/iter; 1.7649x vs baseline; 1.0998x over previous
import jax
import jax.numpy as jnp
from jax import lax
from jax.experimental import pallas as pl
from jax.experimental.pallas import tpu as pltpu

N_DEV = 16
SQ = 256
SKV = 4096
H_LOC = 8
DH = 128
D_MODEL = 1024
D_LOC = H_LOC * DH
CHUNK = SQ // N_DEV
SCALE = 0.08838834764831843


def kernel(x, Wq, K_ext, V_ext, Wo):
    i = lax.axis_index("i")
    x2 = x[0].astype(jnp.bfloat16)
    Wq_s = lax.dynamic_slice(Wq, (0, i * D_LOC), (D_MODEL, D_LOC))
    Wo_s = lax.dynamic_slice(Wo, (i * D_LOC, 0), (D_LOC, D_MODEL))
    Wq_s = (Wq_s * SCALE).astype(jnp.bfloat16)
    Wo_s = Wo_s.astype(jnp.bfloat16)
    K = K_ext[0].reshape(SKV, H_LOC * DH).astype(jnp.bfloat16)
    V = V_ext[0].reshape(SKV, H_LOC * DH).astype(jnp.bfloat16)

    def body(x_ref, wq_ref, k_ref, v_ref, wo_ref, out_ref,
             acc_ref, cacc_ref, rs_buf,
             rs_send, rs_recv, ag_send, ag_recv):
        my = lax.axis_index("i")

        q = jnp.dot(x_ref[...], wq_ref[...],
                    preferred_element_type=jnp.float32)
        qb = lax.broadcasted_iota(jnp.int32, (SQ, SKV), 0) // 64
        kb = lax.broadcasted_iota(jnp.int32, (SQ, SKV), 1) // 64
        mask = (qb == kb) | (kb == 0) | ((qb + kb) % 3 == 0)
        ctx_parts = []
        for h in range(H_LOC):
            q_h = q[:, h * DH:(h + 1) * DH].astype(jnp.bfloat16)
            k_h = k_ref[:, h * DH:(h + 1) * DH]
            s = lax.dot_general(q_h, k_h, (((1,), (1,)), ((), ())),
                                preferred_element_type=jnp.float32)
            w = jnp.where(mask, jnp.exp(s), 0.0)
            denom = jnp.sum(w, axis=-1, keepdims=True)
            v_h = v_ref[:, h * DH:(h + 1) * DH]
            ctx_h = jnp.dot(w.astype(jnp.bfloat16), v_h,
                            preferred_element_type=jnp.float32)
            ctx_parts.append(ctx_h / denom)
        ctx = jnp.concatenate(ctx_parts, axis=-1).astype(jnp.bfloat16)
        partial = jnp.dot(ctx, wo_ref[...],
                          preferred_element_type=jnp.float32)
        acc_ref[...] = partial.reshape(N_DEV, CHUNK, D_MODEL)

        stage_off = 0
        for k, b in enumerate((8, 4, 2, 1)):
            partner = my ^ b
            keep_start = my & (N_DEV - b)
            send_start = keep_start ^ b
            cacc_ref[pl.ds(send_start, b)] = (
                acc_ref[pl.ds(send_start, b)].astype(jnp.bfloat16))
            rdma = pltpu.make_async_remote_copy(
                src_ref=cacc_ref.at[pl.ds(send_start, b)],
                dst_ref=rs_buf.at[pl.ds(stage_off, b)],
                send_sem=rs_send.at[k],
                recv_sem=rs_recv.at[k],
                device_id=(partner,),
                device_id_type=pl.DeviceIdType.MESH,
            )
            rdma.start()
            rdma.wait()
            acc_ref[pl.ds(keep_start, b)] = (
                acc_ref[pl.ds(keep_start, b)]
                + rs_buf[pl.ds(stage_off, b)].astype(jnp.float32)
            )
            stage_off += b

        cacc_ref[pl.ds(my, 1)] = acc_ref[pl.ds(my, 1)].astype(jnp.bfloat16)
        for k, b in enumerate((1, 2, 4, 8)):
            partner = my ^ b
            seg_start = my & (N_DEV - b)
            rdma = pltpu.make_async_remote_copy(
                src_ref=cacc_ref.at[pl.ds(seg_start, b)],
                dst_ref=cacc_ref.at[pl.ds(seg_start, b)],
                send_sem=ag_send.at[k],
                recv_sem=ag_recv.at[k],
                device_id=(partner,),
                device_id_type=pl.DeviceIdType.MESH,
            )
            rdma.start()
            rdma.wait()

        out_ref[...] = cacc_ref[...].astype(jnp.float32).reshape(1, SQ, D_MODEL)

    return pl.pallas_call(
        body,
        out_shape=jax.ShapeDtypeStruct((1, SQ, D_MODEL), jnp.float32),
        in_specs=[pl.BlockSpec(memory_space=pltpu.VMEM)] * 5,
        out_specs=pl.BlockSpec(memory_space=pltpu.VMEM),
        scratch_shapes=[
            pltpu.VMEM((N_DEV, CHUNK, D_MODEL), jnp.float32),
            pltpu.VMEM((N_DEV, CHUNK, D_MODEL), jnp.bfloat16),
            pltpu.VMEM((N_DEV - 1, CHUNK, D_MODEL), jnp.bfloat16),
            pltpu.SemaphoreType.DMA((4,)),
            pltpu.SemaphoreType.DMA((4,)),
            pltpu.SemaphoreType.DMA((4,)),
            pltpu.SemaphoreType.DMA((4,)),
        ],
        compiler_params=pltpu.CompilerParams(
            vmem_limit_bytes=100 * 1024 * 1024,
        ),
    )(x2, Wq_s, K, V, Wo_s)


# device time: 79039 ns/iter; 2.1673x vs baseline; 1.2280x over previous
import jax
import jax.numpy as jnp
from jax import lax
from jax.experimental import pallas as pl
from jax.experimental.pallas import tpu as pltpu

N_DEV = 16
SQ = 256
SKV = 4096
H_LOC = 8
DH = 128
D_MODEL = 1024
D_LOC = H_LOC * DH
CHUNK = SQ // N_DEV
SCALE = 0.08838834764831843


def kernel(x, Wq, K_ext, V_ext, Wo):
    def body(x_ref, wq_hbm, k_hbm, v_hbm, wo_hbm, out_ref,
             wq_st, wo_st, k2_ref, v2_ref, kv_stage, acc_ref, cacc_ref,
             rs_buf, wq_sem, wo_sem, kv_sems,
             rs_send, rs_recv, ag_send, ag_recv):
        my = lax.axis_index("i")

        wq_cp = pltpu.make_async_copy(
            wq_hbm.at[:, pl.ds(my * D_LOC, D_LOC)], wq_st, wq_sem)
        wq_cp.start()
        wo_cp = pltpu.make_async_copy(
            wo_hbm.at[pl.ds(my * D_LOC, D_LOC), :], wo_st, wo_sem)
        wo_cp.start()

        def start_k(h):
            cp = pltpu.make_async_copy(
                k_hbm.at[0, :, h, :], kv_stage.at[0], kv_sems.at[0])
            cp.start()
            return cp

        def start_v(h):
            cp = pltpu.make_async_copy(
                v_hbm.at[0, :, h, :], kv_stage.at[1], kv_sems.at[1])
            cp.start()
            return cp

        cp_k = start_k(0)
        cp_v = start_v(0)

        x_bf = (x_ref[...] * SCALE).astype(jnp.bfloat16)
        wq_cp.wait()
        q = jnp.dot(x_bf, wq_st[...].astype(jnp.bfloat16),
                    preferred_element_type=jnp.float32)

        qb = lax.broadcasted_iota(jnp.int32, (SQ, SKV), 0) // 64
        kb = lax.broadcasted_iota(jnp.int32, (SQ, SKV), 1) // 64
        mask = (qb == kb) | (kb == 0) | ((qb + kb) % 3 == 0)

        ctx_parts = []
        for h in range(H_LOC):
            cp_k.wait()
            k2_ref[:, h * DH:(h + 1) * DH] = kv_stage[0].astype(jnp.bfloat16)
            if h + 1 < H_LOC:
                cp_k = start_k(h + 1)
            cp_v.wait()
            v2_ref[:, h * DH:(h + 1) * DH] = kv_stage[1].astype(jnp.bfloat16)
            if h + 1 < H_LOC:
                cp_v = start_v(h + 1)

            q_h = q[:, h * DH:(h + 1) * DH].astype(jnp.bfloat16)
            k_h = k2_ref[:, h * DH:(h + 1) * DH]
            s = lax.dot_general(q_h, k_h, (((1,), (1,)), ((), ())),
                                preferred_element_type=jnp.float32)
            w = jnp.where(mask, jnp.exp(s), 0.0)
            denom = jnp.sum(w, axis=-1, keepdims=True)
            v_h = v2_ref[:, h * DH:(h + 1) * DH]
            ctx_h = jnp.dot(w.astype(jnp.bfloat16), v_h,
                            preferred_element_type=jnp.float32)
            ctx_parts.append(ctx_h / denom)
        ctx = jnp.concatenate(ctx_parts, axis=-1).astype(jnp.bfloat16)
        wo_cp.wait()
        partial = jnp.dot(ctx, wo_st[...].astype(jnp.bfloat16),
                          preferred_element_type=jnp.float32)
        acc_ref[...] = partial.reshape(N_DEV, CHUNK, D_MODEL)

        stage_off = 0
        for k, b in enumerate((8, 4, 2, 1)):
            partner = my ^ b
            keep_start = my & (N_DEV - b)
            send_start = keep_start ^ b
            cacc_ref[pl.ds(send_start, b)] = (
                acc_ref[pl.ds(send_start, b)].astype(jnp.bfloat16))
            rdma = pltpu.make_async_remote_copy(
                src_ref=cacc_ref.at[pl.ds(send_start, b)],
                dst_ref=rs_buf.at[pl.ds(stage_off, b)],
                send_sem=rs_send.at[k],
                recv_sem=rs_recv.at[k],
                device_id=(partner,),
                device_id_type=pl.DeviceIdType.MESH,
            )
            rdma.start()
            rdma.wait()
            acc_ref[pl.ds(keep_start, b)] = (
                acc_ref[pl.ds(keep_start, b)]
                + rs_buf[pl.ds(stage_off, b)].astype(jnp.float32)
            )
            stage_off += b

        cacc_ref[pl.ds(my, 1)] = acc_ref[pl.ds(my, 1)].astype(jnp.bfloat16)
        for k, b in enumerate((1, 2, 4, 8)):
            partner = my ^ b
            seg_start = my & (N_DEV - b)
            rdma = pltpu.make_async_remote_copy(
                src_ref=cacc_ref.at[pl.ds(seg_start, b)],
                dst_ref=cacc_ref.at[pl.ds(seg_start, b)],
                send_sem=ag_send.at[k],
                recv_sem=ag_recv.at[k],
                device_id=(partner,),
                device_id_type=pl.DeviceIdType.MESH,
            )
            rdma.start()
            rdma.wait()

        out_ref[...] = cacc_ref[...].astype(jnp.float32).reshape(1, SQ, D_MODEL)

    return pl.pallas_call(
        body,
        out_shape=jax.ShapeDtypeStruct((1, SQ, D_MODEL), jnp.float32),
        in_specs=[
            pl.BlockSpec(memory_space=pltpu.VMEM),
            pl.BlockSpec(memory_space=pltpu.MemorySpace.HBM),
            pl.BlockSpec(memory_space=pltpu.MemorySpace.HBM),
            pl.BlockSpec(memory_space=pltpu.MemorySpace.HBM),
            pl.BlockSpec(memory_space=pltpu.MemorySpace.HBM),
        ],
        out_specs=pl.BlockSpec(memory_space=pltpu.VMEM),
        scratch_shapes=[
            pltpu.VMEM((D_MODEL, D_LOC), jnp.float32),
            pltpu.VMEM((D_LOC, D_MODEL), jnp.float32),
            pltpu.VMEM((SKV, H_LOC * DH), jnp.bfloat16),
            pltpu.VMEM((SKV, H_LOC * DH), jnp.bfloat16),
            pltpu.VMEM((2, SKV, DH), jnp.float32),
            pltpu.VMEM((N_DEV, CHUNK, D_MODEL), jnp.float32),
            pltpu.VMEM((N_DEV, CHUNK, D_MODEL), jnp.bfloat16),
            pltpu.VMEM((N_DEV - 1, CHUNK, D_MODEL), jnp.bfloat16),
            pltpu.SemaphoreType.DMA,
            pltpu.SemaphoreType.DMA,
            pltpu.SemaphoreType.DMA((2,)),
            pltpu.SemaphoreType.DMA((4,)),
            pltpu.SemaphoreType.DMA((4,)),
            pltpu.SemaphoreType.DMA((4,)),
            pltpu.SemaphoreType.DMA((4,)),
        ],
        compiler_params=pltpu.CompilerParams(
            vmem_limit_bytes=100 * 1024 * 1024,
        ),
    )(x[0], Wq, K_ext, V_ext, Wo)


# device time: 64166 ns/iter; 2.6696x vs baseline; 1.2318x over previous
import os

import jax
import jax.numpy as jnp
from jax import lax
from jax.experimental import pallas as pl
from jax.experimental.pallas import tpu as pltpu

_SKIP_COMM = os.environ.get("SKIP_COMM") == "1"

N_DEV = 16
SQ = 256
SKV = 4096
H_LOC = 8
DH = 128
D_MODEL = 1024
D_LOC = H_LOC * DH
CHUNK = SQ // N_DEV
SCALE = 0.08838834764831843


def kernel(x, Wq, K_ext, V_ext, Wo):
    def body(x_ref, wq_hbm, k_hbm, v_hbm, wo_hbm, out_ref,
             wq_st, wo_st, kstage, vstage, ctx_ref, acc_ref, cacc_ref,
             rs_buf, wq_sem, wo_sem, k_sems, v_sems,
             rs_send, rs_recv, ag_send, ag_recv):
        my = lax.axis_index("i")
        partners = [my ^ 8, my ^ 4, my ^ 2, my ^ 1]

        if not _SKIP_COMM:
            barrier_sem = pltpu.get_barrier_semaphore()
            for p in partners:
                pl.semaphore_signal(barrier_sem, inc=1, device_id=(p,),
                                    device_id_type=pl.DeviceIdType.MESH)

        wq_cp = pltpu.make_async_copy(
            wq_hbm.at[:, pl.ds(my * D_LOC, D_LOC)], wq_st, wq_sem)
        wq_cp.start()
        wo_cp = pltpu.make_async_copy(
            wo_hbm.at[pl.ds(my * D_LOC, D_LOC), :], wo_st, wo_sem)
        wo_cp.start()

        def start_k(h):
            cp = pltpu.make_async_copy(
                k_hbm.at[0, :, h, :], kstage.at[h % 2], k_sems.at[h % 2])
            cp.start()
            return cp

        def start_v(h):
            cp = pltpu.make_async_copy(
                v_hbm.at[0, :, h, :], vstage.at[h % 2], v_sems.at[h % 2])
            cp.start()
            return cp

        cp_k = start_k(0)
        cp_v = start_v(0)

        x_bf = (x_ref[...] * SCALE).astype(jnp.bfloat16)
        wq_cp.wait()
        q = jnp.dot(x_bf, wq_st[...].astype(jnp.bfloat16),
                    preferred_element_type=jnp.float32)

        qb = lax.broadcasted_iota(jnp.int32, (SQ, SKV), 0) // 64
        kb = lax.broadcasted_iota(jnp.int32, (SQ, SKV), 1) // 64
        mask = (qb == kb) | (kb == 0) | ((qb + kb) % 3 == 0)

        ctx_parts = []
        for h in range(H_LOC):
            cp_k.wait()
            k_h = kstage[h % 2].astype(jnp.bfloat16)
            if h + 1 < H_LOC:
                cp_k = start_k(h + 1)
            q_h = q[:, h * DH:(h + 1) * DH].astype(jnp.bfloat16)
            s = lax.dot_general(q_h, k_h, (((1,), (1,)), ((), ())),
                                preferred_element_type=jnp.float32)
            w = jnp.where(mask, jnp.exp(s), 0.0)
            denom = jnp.sum(w, axis=-1, keepdims=True)
            cp_v.wait()
            v_h = vstage[h % 2].astype(jnp.bfloat16)
            if h + 1 < H_LOC:
                cp_v = start_v(h + 1)
            ctx_h = jnp.dot(w.astype(jnp.bfloat16), v_h,
                            preferred_element_type=jnp.float32)
            ctx_parts.append(ctx_h / denom)
        ctx_ref[...] = jnp.concatenate(ctx_parts, axis=-1).astype(jnp.bfloat16)
        wo_cp.wait()
        wo_bf = wo_st[...].astype(jnp.bfloat16)

        if _SKIP_COMM:
            partial = jnp.dot(ctx_ref[...], wo_bf,
                              preferred_element_type=jnp.float32)
            out_ref[...] = partial.reshape(1, SQ, D_MODEL)
            return

        keep0 = my & 8
        send0 = keep0 ^ 8
        ctx_s = ctx_ref[pl.ds(send0 * CHUNK, 8 * CHUNK), :]
        part_s = jnp.dot(ctx_s, wo_bf, preferred_element_type=jnp.float32)
        acc_ref[pl.ds(send0, 8)] = part_s.reshape(8, CHUNK, D_MODEL)
        cacc_ref[pl.ds(send0, 8)] = (
            part_s.astype(jnp.bfloat16).reshape(8, CHUNK, D_MODEL))
        pl.semaphore_wait(barrier_sem, 4)
        rdma0 = pltpu.make_async_remote_copy(
            src_ref=cacc_ref.at[pl.ds(send0, 8)],
            dst_ref=rs_buf.at[pl.ds(0, 8)],
            send_sem=rs_send.at[0],
            recv_sem=rs_recv.at[0],
            device_id=(my ^ 8,),
            device_id_type=pl.DeviceIdType.MESH,
        )
        rdma0.start()
        ctx_k = ctx_ref[pl.ds(keep0 * CHUNK, 8 * CHUNK), :]
        part_k = jnp.dot(ctx_k, wo_bf, preferred_element_type=jnp.float32)
        rdma0.wait()
        acc_ref[pl.ds(keep0, 8)] = (
            part_k.reshape(8, CHUNK, D_MODEL)
            + rs_buf[pl.ds(0, 8)].astype(jnp.float32)
        )

        stage_off = 8
        for k, b in ((1, 4), (2, 2), (3, 1)):
            partner = my ^ b
            keep_start = my & (N_DEV - b)
            send_start = keep_start ^ b
            cacc_ref[pl.ds(send_start, b)] = (
                acc_ref[pl.ds(send_start, b)].astype(jnp.bfloat16))
            rdma = pltpu.make_async_remote_copy(
                src_ref=cacc_ref.at[pl.ds(send_start, b)],
                dst_ref=rs_buf.at[pl.ds(stage_off, b)],
                send_sem=rs_send.at[k],
                recv_sem=rs_recv.at[k],
                device_id=(partner,),
                device_id_type=pl.DeviceIdType.MESH,
            )
            rdma.start()
            rdma.wait()
            acc_ref[pl.ds(keep_start, b)] = (
                acc_ref[pl.ds(keep_start, b)]
                + rs_buf[pl.ds(stage_off, b)].astype(jnp.float32)
            )
            stage_off += b

        cacc_ref[pl.ds(my, 1)] = acc_ref[pl.ds(my, 1)].astype(jnp.bfloat16)
        for k, b in enumerate((1, 2, 4, 8)):
            partner = my ^ b
            seg_start = my & (N_DEV - b)
            rdma = pltpu.make_async_remote_copy(
                src_ref=cacc_ref.at[pl.ds(seg_start, b)],
                dst_ref=cacc_ref.at[pl.ds(seg_start, b)],
                send_sem=ag_send.at[k],
                recv_sem=ag_recv.at[k],
                device_id=(partner,),
                device_id_type=pl.DeviceIdType.MESH,
            )
            rdma.start()
            rdma.wait()

        out_ref[...] = cacc_ref[...].astype(jnp.float32).reshape(1, SQ, D_MODEL)

    return pl.pallas_call(
        body,
        out_shape=jax.ShapeDtypeStruct((1, SQ, D_MODEL), jnp.float32),
        in_specs=[
            pl.BlockSpec(memory_space=pltpu.VMEM),
            pl.BlockSpec(memory_space=pltpu.MemorySpace.HBM),
            pl.BlockSpec(memory_space=pltpu.MemorySpace.HBM),
            pl.BlockSpec(memory_space=pltpu.MemorySpace.HBM),
            pl.BlockSpec(memory_space=pltpu.MemorySpace.HBM),
        ],
        out_specs=pl.BlockSpec(memory_space=pltpu.VMEM),
        scratch_shapes=[
            pltpu.VMEM((D_MODEL, D_LOC), jnp.float32),
            pltpu.VMEM((D_LOC, D_MODEL), jnp.float32),
            pltpu.VMEM((2, SKV, DH), jnp.float32),
            pltpu.VMEM((2, SKV, DH), jnp.float32),
            pltpu.VMEM((SQ, D_LOC), jnp.bfloat16),
            pltpu.VMEM((N_DEV, CHUNK, D_MODEL), jnp.float32),
            pltpu.VMEM((N_DEV, CHUNK, D_MODEL), jnp.bfloat16),
            pltpu.VMEM((N_DEV - 1, CHUNK, D_MODEL), jnp.bfloat16),
            pltpu.SemaphoreType.DMA,
            pltpu.SemaphoreType.DMA,
            pltpu.SemaphoreType.DMA((2,)),
            pltpu.SemaphoreType.DMA((2,)),
            pltpu.SemaphoreType.DMA((4,)),
            pltpu.SemaphoreType.DMA((4,)),
            pltpu.SemaphoreType.DMA((4,)),
            pltpu.SemaphoreType.DMA((4,)),
        ],
        compiler_params=pltpu.CompilerParams(
            vmem_limit_bytes=100 * 1024 * 1024,
            **({} if _SKIP_COMM else {"collective_id": 0}),
        ),
    )(x[0], Wq, K_ext, V_ext, Wo)


# device time: 48779 ns/iter; 3.5117x vs baseline; 1.3154x over previous
import os

import jax
import jax.numpy as jnp
from jax import lax
from jax.experimental import pallas as pl
from jax.experimental.pallas import tpu as pltpu

_SKIP_COMM = os.environ.get("SKIP_COMM") == "1"

N_DEV = 16
SQ = 256
SKV = 4096
H_LOC = 8
DH = 128
D_MODEL = 1024
D_LOC = H_LOC * DH
CHUNK = SQ // N_DEV
SCALE = 0.08838834764831843


def kernel(x, Wq, K_ext, V_ext, Wo):
    def body(x_ref, wq_hbm, k_hbm, v_hbm, wo_hbm, out_ref,
             wq_st, wo_st, kstage, vstage, ctx_ref, cacc_ref,
             rs_buf, wq_sem, wo_sem, k_sems, v_sems,
             rs_send, rs_recv, ag_send, ag_recv):
        my = lax.axis_index("i")

        if not _SKIP_COMM:
            barrier_sem = pltpu.get_barrier_semaphore()
            for j in range(N_DEV):
                @pl.when(my != j)
                def _(j=j):
                    pl.semaphore_signal(barrier_sem, inc=1, device_id=(j,),
                                        device_id_type=pl.DeviceIdType.MESH)

        wq_cp = pltpu.make_async_copy(
            wq_hbm.at[:, pl.ds(my * D_LOC, D_LOC)], wq_st, wq_sem)
        wq_cp.start()
        wo_cp = pltpu.make_async_copy(
            wo_hbm.at[pl.ds(my * D_LOC, D_LOC), :], wo_st, wo_sem)
        wo_cp.start()

        def start_k(h):
            cp = pltpu.make_async_copy(
                k_hbm.at[0, :, h, :], kstage.at[h % 2], k_sems.at[h % 2])
            cp.start()
            return cp

        def start_v(h):
            cp = pltpu.make_async_copy(
                v_hbm.at[0, :, h, :], vstage.at[h % 2], v_sems.at[h % 2])
            cp.start()
            return cp

        cp_k = start_k(0)
        cp_v = start_v(0)

        x_bf = (x_ref[...] * SCALE).astype(jnp.bfloat16)
        wq_cp.wait()
        q = jnp.dot(x_bf, wq_st[...].astype(jnp.bfloat16),
                    preferred_element_type=jnp.float32)

        qb = lax.broadcasted_iota(jnp.int32, (SQ, SKV), 0) // 64
        kb = lax.broadcasted_iota(jnp.int32, (SQ, SKV), 1) // 64
        mask = (qb == kb) | (kb == 0) | ((qb + kb) % 3 == 0)

        ctx_parts = []
        for h in range(H_LOC):
            cp_k.wait()
            k_h = kstage[h % 2].astype(jnp.bfloat16)
            if h + 1 < H_LOC:
                cp_k = start_k(h + 1)
            q_h = q[:, h * DH:(h + 1) * DH].astype(jnp.bfloat16)
            s = lax.dot_general(q_h, k_h, (((1,), (1,)), ((), ())),
                                preferred_element_type=jnp.float32)
            w = jnp.where(mask, jnp.exp(s), 0.0)
            denom = jnp.sum(w, axis=-1, keepdims=True)
            cp_v.wait()
            v_h = vstage[h % 2].astype(jnp.bfloat16)
            if h + 1 < H_LOC:
                cp_v = start_v(h + 1)
            ctx_h = jnp.dot(w.astype(jnp.bfloat16), v_h,
                            preferred_element_type=jnp.float32)
            ctx_parts.append(ctx_h / denom)
        ctx_ref[...] = jnp.concatenate(ctx_parts, axis=-1).astype(jnp.bfloat16)
        wo_cp.wait()
        wo_bf = wo_st[...].astype(jnp.bfloat16)

        if _SKIP_COMM:
            partial = jnp.dot(ctx_ref[...], wo_bf,
                              preferred_element_type=jnp.float32)
            out_ref[...] = partial.reshape(1, SQ, D_MODEL)
            return

        partial = jnp.dot(ctx_ref[...], wo_bf,
                          preferred_element_type=jnp.float32)
        cacc_ref[...] = (
            partial.astype(jnp.bfloat16).reshape(N_DEV, CHUNK, D_MODEL))
        pl.semaphore_wait(barrier_sem, N_DEV - 1)

        def a2a(src_slot_fn, buf, send_sems, recv_sems):
            descs = []
            for j in range(N_DEV):
                rdma = pltpu.make_async_remote_copy(
                    src_ref=cacc_ref.at[src_slot_fn(j)],
                    dst_ref=buf.at[my],
                    send_sem=send_sems.at[j],
                    recv_sem=recv_sems.at[my],
                    device_id=(j,),
                    device_id_type=pl.DeviceIdType.MESH,
                )
                @pl.when(my != j)
                def _(rdma=rdma):
                    rdma.start()
                descs.append(rdma)
            return descs

        rs_descs = a2a(lambda j: j, rs_buf, rs_send, rs_recv)

        rs_buf[pl.ds(my, 1)] = cacc_ref[pl.ds(my, 1)]

        for j in range(N_DEV):
            rdma = pltpu.make_async_remote_copy(
                src_ref=cacc_ref.at[j],
                dst_ref=rs_buf.at[j],
                send_sem=rs_send.at[j],
                recv_sem=rs_recv.at[j],
                device_id=(j,),
                device_id_type=pl.DeviceIdType.MESH,
            )
            @pl.when(my != j)
            def _(rdma=rdma):
                rdma.wait_recv()

        red = jnp.sum(rs_buf[...].astype(jnp.float32), axis=0)
        cacc_ref[pl.ds(my, 1)] = (
            red.astype(jnp.bfloat16).reshape(1, CHUNK, D_MODEL))

        ag_descs = a2a(lambda j: my, cacc_ref, ag_send, ag_recv)

        for j in range(N_DEV):
            rdma = pltpu.make_async_remote_copy(
                src_ref=cacc_ref.at[j],
                dst_ref=cacc_ref.at[j],
                send_sem=ag_send.at[j],
                recv_sem=ag_recv.at[j],
                device_id=(j,),
                device_id_type=pl.DeviceIdType.MESH,
            )
            @pl.when(my != j)
            def _(rdma=rdma):
                rdma.wait_recv()

        for descs in (rs_descs, ag_descs):
            for j, rdma in enumerate(descs):
                @pl.when(my != j)
                def _(rdma=rdma):
                    rdma.wait_send()

        out_ref[...] = cacc_ref[...].astype(jnp.float32).reshape(1, SQ, D_MODEL)

    return pl.pallas_call(
        body,
        out_shape=jax.ShapeDtypeStruct((1, SQ, D_MODEL), jnp.float32),
        in_specs=[
            pl.BlockSpec(memory_space=pltpu.VMEM),
            pl.BlockSpec(memory_space=pltpu.MemorySpace.HBM),
            pl.BlockSpec(memory_space=pltpu.MemorySpace.HBM),
            pl.BlockSpec(memory_space=pltpu.MemorySpace.HBM),
            pl.BlockSpec(memory_space=pltpu.MemorySpace.HBM),
        ],
        out_specs=pl.BlockSpec(memory_space=pltpu.VMEM),
        scratch_shapes=[
            pltpu.VMEM((D_MODEL, D_LOC), jnp.float32),
            pltpu.VMEM((D_LOC, D_MODEL), jnp.float32),
            pltpu.VMEM((2, SKV, DH), jnp.float32),
            pltpu.VMEM((2, SKV, DH), jnp.float32),
            pltpu.VMEM((SQ, D_LOC), jnp.bfloat16),
            pltpu.VMEM((N_DEV, CHUNK, D_MODEL), jnp.bfloat16),
            pltpu.VMEM((N_DEV, CHUNK, D_MODEL), jnp.bfloat16),
            pltpu.SemaphoreType.DMA,
            pltpu.SemaphoreType.DMA,
            pltpu.SemaphoreType.DMA((2,)),
            pltpu.SemaphoreType.DMA((2,)),
            pltpu.SemaphoreType.DMA((N_DEV,)),
            pltpu.SemaphoreType.DMA((N_DEV,)),
            pltpu.SemaphoreType.DMA((N_DEV,)),
            pltpu.SemaphoreType.DMA((N_DEV,)),
        ],
        compiler_params=pltpu.CompilerParams(
            vmem_limit_bytes=100 * 1024 * 1024,
            **({} if _SKIP_COMM else {"collective_id": 0}),
        ),
    )(x[0], Wq, K_ext, V_ext, Wo)
